# Initial kernel scaffold; baseline (speedup 1.0000x reference)
import jax
import jax.numpy as jnp
from jax.experimental import pallas as pl


def kernel(x, gate_weight, expert_bias, W1, b1, W2, b2):
    raise NotImplementedError("write your pallas kernel here")


# speedup vs baseline: 1.0621x; 1.0621x over previous
import functools

import jax
import jax.numpy as jnp
from jax.experimental import pallas as pl
from jax.experimental.pallas import tpu as pltpu

_CAPACITY = 2.0


def _gelu_tanh(v):
    return 0.5 * v * (1.0 + jnp.tanh(jnp.sqrt(2.0 / jnp.pi) * (v + 0.044715 * v ** 3)))


def _router_kernel(x_ref, gw_ref, bias_ref, comb_ref, stats_ref, *, k):
    x = x_ref[...]
    gw = gw_ref[...]
    bias = bias_ref[...]
    S = x.shape[0]
    E = gw.shape[0]
    logits = jax.lax.dot_general(x, gw, (((1,), (1,)), ((), ())),
                                 preferred_element_type=jnp.float32)
    scores = jax.nn.sigmoid(logits + bias[:, 0][None, :])
    si = jax.lax.bitcast_convert_type(scores, jnp.int32)

    def _tstep(i, t):
        cand = t | (jnp.int32(1) << (30 - i))
        cnt = jnp.sum((si >= cand).astype(jnp.int32), dtype=jnp.int32)
        return jnp.where(cnt >= k, cand, t)

    t = jax.lax.fori_loop(0, 31, _tstep, jnp.int32(0))

    gt = si > t
    eq = si == t
    cg = jnp.sum(gt.astype(jnp.int32), dtype=jnp.int32)
    need = k - cg

    s_iota = jax.lax.broadcasted_iota(jnp.int32, (S, E), 0)
    e_iota = jax.lax.broadcasted_iota(jnp.int32, (S, E), 1)
    fidx = e_iota * S + s_iota

    def _mstep(_, lohi):
        lo, hi = lohi
        mid = (lo + hi) // 2
        cnt = jnp.sum((eq & (fidx <= mid)).astype(jnp.int32), dtype=jnp.int32)
        return jnp.where(cnt >= need, lo, mid + 1), jnp.where(cnt >= need, mid, hi)

    lo, _hi = jax.lax.fori_loop(0, 14, _mstep,
                                (jnp.int32(0), jnp.int32(E * S - 1)))

    sel = gt | (eq & (fidx <= lo))
    comb_ref[...] = jnp.where(sel, scores, 0.0)
    counts = jnp.sum(sel.astype(jnp.float32), axis=0)
    stats_ref[...] = (counts / float(k))[:, None] * jnp.ones((E, 128), jnp.float32)


def _router_call(x_flat, gate_weight, expert_bias, k):
    S, D = x_flat.shape
    E = gate_weight.shape[0]
    return pl.pallas_call(
        functools.partial(_router_kernel, k=k),
        out_shape=(
            jax.ShapeDtypeStruct((S, E), jnp.float32),
            jax.ShapeDtypeStruct((E, 128), jnp.float32),
        ),
    )(x_flat, gate_weight, expert_bias)


def _experts_kernel(x_ref, w1_ref, b1_ref, w2_ref, b2_ref, comb_ref, out_ref):
    e = pl.program_id(0)
    f = pl.program_id(1)

    @pl.when((e == 0) & (f == 0))
    def _():
        out_ref[...] = jnp.zeros_like(out_ref)

    x = x_ref[...]
    w1 = w1_ref[0]
    b1 = b1_ref[0]
    w2 = w2_ref[0]
    comb = comb_ref[0]

    h = jax.lax.dot_general(x, w1, (((1,), (1,)), ((), ())),
                            preferred_element_type=jnp.float32)
    h = _gelu_tanh(h + b1)
    y = jax.lax.dot_general(h, w2, (((1,), (1,)), ((), ())),
                            preferred_element_type=jnp.float32)

    @pl.when(f == 0)
    def _():
        out_ref[...] += comb * b2_ref[0]

    out_ref[...] += comb * y


def _experts_call(x_flat, W1, b1, W2, b2, comb3):
    S, D = x_flat.shape
    E, DFF, _ = W1.shape
    FT = 768
    F = DFF // FT
    return pl.pallas_call(
        _experts_kernel,
        grid=(E, F),
        in_specs=[
            pl.BlockSpec((S, D), lambda e, f: (0, 0)),
            pl.BlockSpec((1, FT, D), lambda e, f: (e, f, 0)),
            pl.BlockSpec((1, 1, FT), lambda e, f: (e * F + f, 0, 0)),
            pl.BlockSpec((1, D, FT), lambda e, f: (e, 0, f)),
            pl.BlockSpec((1, 1, D), lambda e, f: (e, 0, 0)),
            pl.BlockSpec((1, S, 1), lambda e, f: (e, 0, 0)),
        ],
        out_specs=pl.BlockSpec((S, D), lambda e, f: (0, 0)),
        out_shape=jax.ShapeDtypeStruct((S, D), jnp.float32),
    )(x_flat, W1, b1.reshape(E * F, 1, FT), W2, b2.reshape(E, 1, D), comb3)


def kernel(x, gate_weight, expert_bias, W1, b1, W2, b2):
    Bsz, seq, D = x.shape
    E = gate_weight.shape[0]
    x_flat = x.reshape(-1, D)
    S = x_flat.shape[0]
    k = int(S * _CAPACITY)

    comb_se, stats = _router_call(x_flat, gate_weight, expert_bias, k)
    comb3 = comb_se.T.reshape(E, S, 1)
    out = _experts_call(x_flat, W1, b1, W2, b2, comb3)

    x_out = out.reshape(Bsz, seq, D)
    token_each_expert = stats[:, 0]
    ones_like_mean = jnp.ones((E,), dtype=x.dtype)
    return (x_out, token_each_expert, ones_like_mean)
